# baseline (speedup 1.0000x reference)
import functools

import jax
import jax.numpy as jnp
from jax import lax
from jax.experimental import pallas as pl
from jax.experimental.pallas import tpu as pltpu
from jax.experimental.pallas import tpu_sc as plsc

_N = 50000
_D = 128
_DEG = 16
_B = 25000
_E = 128
_NC = 2
_NS = 16
_NW = _NC * _NS
_BP = 25088
_PW = _BP // _NW
_C = 112
_NCH = _PW // _C


def _sc_body(nodes_r, feats_r, n0_r, n1_r,
             selfs_r, sum0_r, sum1_r,
             idxv, idxv8, perm, nbbuf, nbt0, nbt1, selfbuf,
             rbufs, accs, gsems, ssems,
             sem_self, sem_n0, sem_o):
    cid = lax.axis_index("c")
    sid = lax.axis_index("s")
    wid = sid * _NC + cid
    base = wid * _PW

    iota16 = lax.iota(jnp.int32, 16)

    def transpose_nb(nbt):
        for t in range(_C // 16):
            iv = idxv[pl.ds(16 * t, 16)]
            lanebase = (iv & 7) * 16
            rows = iota16 + (16 * t)
            for j in range(_DEG):
                v = plsc.load_gather(nbbuf, [rows, lanebase + j])
                nbt[j, pl.ds(16 * t, 16)] = v

    _NB = 4
    _LOOKAHEAD = _NB - 1
    _NS2 = 2 * _DEG

    for t in range(_C // 16):
        perm[pl.ds(16 * t, 16)] = iota16 + 16 * t

    def chunk_body(k, carry):
        rowbase = base + k * _C
        pltpu.sync_copy(nodes_r.at[pl.ds(rowbase, _C)], idxv)
        for t in range(_C // 16):
            idxv8[pl.ds(16 * t, 16)] = idxv[pl.ds(16 * t, 16)] >> 3
        cp_self = pltpu.async_copy(feats_r.at[idxv], selfbuf, sem_self)
        cp_n = pltpu.async_copy(n0_r.at[idxv8], nbbuf, sem_n0)
        cp_n.wait()
        transpose_nb(nbt0)
        cp_n = pltpu.async_copy(n1_r.at[idxv8], nbbuf, sem_n0)

        def nbt_at(s):
            return (nbt0 if s < _DEG else nbt1).at[s % _DEG]

        def gather(s):
            return pltpu.async_copy(feats_r.at[nbt_at(s)], rbufs[s % _NB],
                                    gsems[s % _NB])

        cps = [None] * _NS2
        for s in range(_LOOKAHEAD):
            cps[s] = gather(s)
        cp_n.wait()
        transpose_nb(nbt1)
        sc_cps = [None] * _NS2
        for s in range(_NS2):
            if s > 0:
                sc_cps[s - 1].wait()
            if s + _LOOKAHEAD < _NS2:
                cps[s + _LOOKAHEAD] = gather(s + _LOOKAHEAD)
            cps[s].wait()
            r = 0 if s < _DEG else 1
            sc_cps[s] = pltpu.async_copy(
                rbufs[s % _NB], accs.at[sid, r].at[perm], ssems[s % _NB],
                add=(s % _DEG != 0))
        sc_cps[_NS2 - 1].wait()
        cp_o0 = pltpu.async_copy(accs.at[sid, 0], sum0_r.at[pl.ds(rowbase, _C)],
                                 sem_o)
        cp_o1 = pltpu.async_copy(accs.at[sid, 1], sum1_r.at[pl.ds(rowbase, _C)],
                                 sem_o)
        cp_self.wait()
        cp_os = pltpu.async_copy(selfbuf, selfs_r.at[pl.ds(rowbase, _C)], sem_o)
        cp_o0.wait()
        cp_o1.wait()
        cp_os.wait()
        return carry

    lax.fori_loop(0, _NCH, chunk_body, 0)


_sc_gather = pl.kernel(
    _sc_body,
    out_type=(
        jax.ShapeDtypeStruct((_BP, _D), jnp.float32),
        jax.ShapeDtypeStruct((_BP, _D), jnp.float32),
        jax.ShapeDtypeStruct((_BP, _D), jnp.float32),
    ),
    mesh=plsc.VectorSubcoreMesh(
        core_axis_name="c", subcore_axis_name="s",
        num_cores=_NC, num_subcores=_NS),
    compiler_params=pltpu.CompilerParams(needs_layout_passes=False),
    scratch_types=[
        pltpu.VMEM((_C,), jnp.int32),
        pltpu.VMEM((_C,), jnp.int32),
        pltpu.VMEM((_C,), jnp.int32),
        pltpu.VMEM((_C, _D), jnp.int32),
        pltpu.VMEM((_DEG, _C), jnp.int32),
        pltpu.VMEM((_DEG, _C), jnp.int32),
        pltpu.VMEM((_C, _D), jnp.float32),
        tuple(pltpu.VMEM((_C, _D), jnp.float32) for _ in range(4)),
        pltpu.VMEM_SHARED((_NS, 2, _C, _D), jnp.float32),
        tuple(pltpu.SemaphoreType.DMA for _ in range(4)),
        tuple(pltpu.SemaphoreType.DMA for _ in range(4)),
        pltpu.SemaphoreType.DMA,
        pltpu.SemaphoreType.DMA,
        pltpu.SemaphoreType.DMA,
    ],
)


def _mlp_body(xs_ref, x0_ref, x1_ref, w1_ref, b1_ref, w2_ref, b2_ref, o_ref):
    dot = functools.partial(jnp.dot, preferred_element_type=jnp.float32,
                            precision=lax.Precision.DEFAULT)
    xs = xs_ref[...]
    x0 = x0_ref[...] * (1.0 / _DEG)
    x1 = x1_ref[...] * (1.0 / _DEG)
    w1 = w1_ref[...]
    acc = dot(xs, w1[0:_D])
    acc = acc + dot(x0, w1[_D:2 * _D])
    acc = acc + dot(x1, w1[2 * _D:3 * _D])
    h = jnp.tanh(acc + b1_ref[...])
    o_ref[...] = dot(h, w2_ref[...]) + b2_ref[...]


_RB = 200


def _mlp(xs, x0, x1, W1, b1, W2, b2):
    grid = (_B // _RB,)
    blk = lambda i: (i, 0)
    full = lambda i: (0, 0)
    return pl.pallas_call(
        _mlp_body,
        grid=grid,
        in_specs=[
            pl.BlockSpec((_RB, _D), blk),
            pl.BlockSpec((_RB, _D), blk),
            pl.BlockSpec((_RB, _D), blk),
            pl.BlockSpec((3 * _D, _D), full),
            pl.BlockSpec((1, _D), full),
            pl.BlockSpec((_D, _E), full),
            pl.BlockSpec((1, _E), full),
        ],
        out_specs=pl.BlockSpec((_RB, _E), blk),
        out_shape=jax.ShapeDtypeStruct((_B, _E), jnp.float32),
    )(xs, x0, x1, W1, b1, W2, b2)


def kernel(nodes, features, neigh_idx_0, neigh_idx_1, W1, b1, W2, b2):
    n0r = neigh_idx_0.reshape(_N * _DEG // _D, _D)
    n1r = neigh_idx_1.reshape(_N * _DEG // _D, _D)
    nodes_p = jnp.zeros((_BP,), jnp.int32).at[:_B].set(nodes)
    selfs, sum0, sum1 = _sc_gather(nodes_p, features, n0r, n1r)
    return _mlp(selfs, sum0, sum1, W1, b1.reshape(1, _D), W2, b2.reshape(1, _E))


# speedup vs baseline: 1.0437x; 1.0437x over previous
import functools

import jax
import jax.numpy as jnp
from jax import lax
from jax.experimental import pallas as pl
from jax.experimental.pallas import tpu as pltpu
from jax.experimental.pallas import tpu_sc as plsc

_N = 50000
_D = 128
_DEG = 16
_B = 25000
_E = 128
_NC = 2
_NS = 16
_NW = _NC * _NS
_BP = 25088
_PW = _BP // _NW
_C = 112
_NCH = _PW // _C


def _sc_body(nodes_r, feats_r, n0_r, n1_r,
             selfs_r, sum0_r, sum1_r,
             idxa, idx8a, idxb, idx8b, perm, nbbuf,
             nbta0, nbta1, nbtb0, nbtb1, selfbuf,
             rbufs, accs, gsems, ssems,
             sem_self, sem_n0, sem_idx, sem_o):
    cid = lax.axis_index("c")
    sid = lax.axis_index("s")
    wid = sid * _NC + cid
    base = wid * _PW

    iota16 = lax.iota(jnp.int32, 16)

    def compute_idx8(idxv, idxv8):
        for t in range(_C // 16):
            idxv8[pl.ds(16 * t, 16)] = idxv[pl.ds(16 * t, 16)] >> 3

    def transpose_nb(idxv, nbt):
        for t in range(_C // 16):
            iv = idxv[pl.ds(16 * t, 16)]
            lanebase = (iv & 7) * 16
            rows = iota16 + (16 * t)
            for j in range(_DEG):
                v = plsc.load_gather(nbbuf, [rows, lanebase + j])
                nbt[j, pl.ds(16 * t, 16)] = v

    _NB = 4
    _LOOKAHEAD = _NB - 1
    _NS2 = 2 * _DEG

    for t in range(_C // 16):
        perm[pl.ds(16 * t, 16)] = iota16 + 16 * t

    def head(k, idxv, idxv8, nbt0, nbt1):
        rowbase = base + k * _C
        pltpu.sync_copy(nodes_r.at[pl.ds(rowbase, _C)], idxv)
        compute_idx8(idxv, idxv8)
        pltpu.sync_copy(n0_r.at[idxv8], nbbuf)
        transpose_nb(idxv, nbt0)
        cp_n = pltpu.async_copy(n1_r.at[idxv8], nbbuf, sem_n0)
        cp_n.wait()
        transpose_nb(idxv, nbt1)

    def steps(k, idxv, nbt0, nbt1, nidxv, nidx8, nnbt0, nnbt1, prefetch):
        rowbase = base + k * _C
        cp_self = pltpu.async_copy(feats_r.at[idxv], selfbuf, sem_self)
        if prefetch:
            nrowbase = base + (k + 1) * _C
            cp_idx = pltpu.async_copy(nodes_r.at[pl.ds(nrowbase, _C)], nidxv,
                                      sem_idx)

        def nbt_at(s):
            return (nbt0 if s < _DEG else nbt1).at[s % _DEG]

        def gather(s):
            return pltpu.async_copy(feats_r.at[nbt_at(s)], rbufs[s % _NB],
                                    gsems[s % _NB])

        cps = [None] * _NS2
        for s in range(_LOOKAHEAD):
            cps[s] = gather(s)
        sc_cps = [None] * _NS2
        cp_n = None
        for s in range(_NS2):
            if s > 0:
                sc_cps[s - 1].wait()
            if s + _LOOKAHEAD < _NS2:
                cps[s + _LOOKAHEAD] = gather(s + _LOOKAHEAD)
            if prefetch and s == 0:
                cp_idx.wait()
                compute_idx8(nidxv, nidx8)
                cp_n = pltpu.async_copy(n0_r.at[nidx8], nbbuf, sem_n0)
            if prefetch and s == 6:
                cp_n.wait()
                transpose_nb(nidxv, nnbt0)
                cp_n = pltpu.async_copy(n1_r.at[nidx8], nbbuf, sem_n0)
            if prefetch and s == 12:
                cp_n.wait()
                transpose_nb(nidxv, nnbt1)
            cps[s].wait()
            r = 0 if s < _DEG else 1
            sc_cps[s] = pltpu.async_copy(
                rbufs[s % _NB], accs.at[sid, r].at[perm], ssems[s % _NB],
                add=(s % _DEG != 0))
        sc_cps[_NS2 - 1].wait()
        cp_o0 = pltpu.async_copy(accs.at[sid, 0], sum0_r.at[pl.ds(rowbase, _C)],
                                 sem_o)
        cp_o1 = pltpu.async_copy(accs.at[sid, 1], sum1_r.at[pl.ds(rowbase, _C)],
                                 sem_o)
        cp_self.wait()
        cp_os = pltpu.async_copy(selfbuf, selfs_r.at[pl.ds(rowbase, _C)], sem_o)
        cp_o0.wait()
        cp_o1.wait()
        cp_os.wait()

    bufa = (idxa, nbta0, nbta1)
    bufb = (idxb, nbtb0, nbtb1)
    head(0, idxa, idx8a, nbta0, nbta1)

    def pair_body(m, carry):
        k = 2 * m
        steps(k, *bufa, idxb, idx8b, nbtb0, nbtb1, prefetch=True)
        steps(k + 1, *bufb, idxa, idx8a, nbta0, nbta1, prefetch=True)
        return carry

    lax.fori_loop(0, (_NCH - 1) // 2, pair_body, 0)
    steps(_NCH - 1, *bufa, idxb, idx8b, nbtb0, nbtb1, prefetch=False)


_sc_gather = pl.kernel(
    _sc_body,
    out_type=(
        jax.ShapeDtypeStruct((_BP, _D), jnp.float32),
        jax.ShapeDtypeStruct((_BP, _D), jnp.float32),
        jax.ShapeDtypeStruct((_BP, _D), jnp.float32),
    ),
    mesh=plsc.VectorSubcoreMesh(
        core_axis_name="c", subcore_axis_name="s",
        num_cores=_NC, num_subcores=_NS),
    compiler_params=pltpu.CompilerParams(needs_layout_passes=False),
    scratch_types=[
        pltpu.VMEM((_C,), jnp.int32),
        pltpu.VMEM((_C,), jnp.int32),
        pltpu.VMEM((_C,), jnp.int32),
        pltpu.VMEM((_C,), jnp.int32),
        pltpu.VMEM((_C,), jnp.int32),
        pltpu.VMEM((_C, _D), jnp.int32),
        pltpu.VMEM((_DEG, _C), jnp.int32),
        pltpu.VMEM((_DEG, _C), jnp.int32),
        pltpu.VMEM((_DEG, _C), jnp.int32),
        pltpu.VMEM((_DEG, _C), jnp.int32),
        pltpu.VMEM((_C, _D), jnp.float32),
        tuple(pltpu.VMEM((_C, _D), jnp.float32) for _ in range(4)),
        pltpu.VMEM_SHARED((_NS, 2, _C, _D), jnp.float32),
        tuple(pltpu.SemaphoreType.DMA for _ in range(4)),
        tuple(pltpu.SemaphoreType.DMA for _ in range(4)),
        pltpu.SemaphoreType.DMA,
        pltpu.SemaphoreType.DMA,
        pltpu.SemaphoreType.DMA,
        pltpu.SemaphoreType.DMA,
    ],
)


def _mlp_body(xs_ref, x0_ref, x1_ref, w1_ref, b1_ref, w2_ref, b2_ref, o_ref):
    dot = functools.partial(jnp.dot, preferred_element_type=jnp.float32,
                            precision=lax.Precision.DEFAULT)
    xs = xs_ref[...]
    x0 = x0_ref[...] * (1.0 / _DEG)
    x1 = x1_ref[...] * (1.0 / _DEG)
    w1 = w1_ref[...]
    acc = dot(xs, w1[0:_D])
    acc = acc + dot(x0, w1[_D:2 * _D])
    acc = acc + dot(x1, w1[2 * _D:3 * _D])
    h = jnp.tanh(acc + b1_ref[...])
    o_ref[...] = dot(h, w2_ref[...]) + b2_ref[...]


_RB = 256


def _mlp(xs, x0, x1, W1, b1, W2, b2):
    grid = (_BP // _RB,)
    blk = lambda i: (i, 0)
    full = lambda i: (0, 0)
    return pl.pallas_call(
        _mlp_body,
        grid=grid,
        in_specs=[
            pl.BlockSpec((_RB, _D), blk),
            pl.BlockSpec((_RB, _D), blk),
            pl.BlockSpec((_RB, _D), blk),
            pl.BlockSpec((3 * _D, _D), full),
            pl.BlockSpec((1, _D), full),
            pl.BlockSpec((_D, _E), full),
            pl.BlockSpec((1, _E), full),
        ],
        out_specs=pl.BlockSpec((_RB, _E), blk),
        out_shape=jax.ShapeDtypeStruct((_BP, _E), jnp.float32),
    )(xs, x0, x1, W1, b1, W2, b2)


def kernel(nodes, features, neigh_idx_0, neigh_idx_1, W1, b1, W2, b2):
    n0r = neigh_idx_0.reshape(_N * _DEG // _D, _D)
    n1r = neigh_idx_1.reshape(_N * _DEG // _D, _D)
    nodes_p = jnp.zeros((_BP,), jnp.int32).at[:_B].set(nodes)
    selfs, sum0, sum1 = _sc_gather(nodes_p, features, n0r, n1r)
    out = _mlp(selfs, sum0, sum1, W1, b1.reshape(1, _D), W2, b2.reshape(1, _E))
    return out[:_B]


# speedup vs baseline: 1.1935x; 1.1435x over previous
import functools

import jax
import jax.numpy as jnp
from jax import lax
from jax.experimental import pallas as pl
from jax.experimental.pallas import tpu as pltpu
from jax.experimental.pallas import tpu_sc as plsc

_N = 50000
_D = 128
_DEG = 16
_B = 25000
_E = 128
_NC = 2
_NS = 16
_NW = _NC * _NS
_BP = 25088
_PW = _BP // _NW
_C = 112
_NCH = _PW // _C


def _sc_body(nodes_r, feats_r, n0_r, n1_r,
             selfs_r, sum0_r, sum1_r,
             idxa, idx8a, idxb, idx8b, perm, nbbuf,
             nbta0, nbta1, nbtb0, nbtb1, selfbuf,
             rbufs, accs, gsems, ssems,
             sem_self, sem_n0, sem_idx, sem_o):
    cid = lax.axis_index("c")
    sid = lax.axis_index("s")
    wid = sid * _NC + cid
    base = wid * _PW

    iota16 = lax.iota(jnp.int32, 16)

    def compute_idx8(idxv, idxv8):
        for t in range(_C // 16):
            idxv8[pl.ds(16 * t, 16)] = idxv[pl.ds(16 * t, 16)] >> 3

    def transpose_nb(idxv, nbt):
        for t in range(_C // 16):
            iv = idxv[pl.ds(16 * t, 16)]
            lanebase = (iv & 7) * 16
            rows = iota16 + (16 * t)
            for j in range(_DEG):
                v = plsc.load_gather(nbbuf, [rows, lanebase + j])
                nbt[j, pl.ds(16 * t, 16)] = v

    _NB = 4
    _LOOKAHEAD = _NB - 1
    _NS2 = 2 * _DEG

    for t in range(_C // 16):
        perm[pl.ds(16 * t, 16)] = iota16 + 16 * t

    def head(k, idxv, idxv8, nbt0, nbt1):
        rowbase = jnp.minimum(base + k * _C, _B - _C)
        pltpu.sync_copy(nodes_r.at[pl.ds(rowbase, _C)], idxv)
        compute_idx8(idxv, idxv8)
        pltpu.sync_copy(n0_r.at[idxv8], nbbuf)
        transpose_nb(idxv, nbt0)
        cp_n = pltpu.async_copy(n1_r.at[idxv8], nbbuf, sem_n0)
        cp_n.wait()
        transpose_nb(idxv, nbt1)

    def steps(k, idxv, nbt0, nbt1, nidxv, nidx8, nnbt0, nnbt1, prefetch):
        rowbase = jnp.minimum(base + k * _C, _B - _C)
        cp_self = pltpu.async_copy(feats_r.at[idxv], selfbuf, sem_self)
        if prefetch:
            nrowbase = jnp.minimum(base + (k + 1) * _C, _B - _C)
            cp_idx = pltpu.async_copy(nodes_r.at[pl.ds(nrowbase, _C)], nidxv,
                                      sem_idx)

        def nbt_at(s):
            return (nbt0 if s < _DEG else nbt1).at[s % _DEG]

        def gather(s):
            return pltpu.async_copy(feats_r.at[nbt_at(s)], rbufs[s % _NB],
                                    gsems[s % _NB])

        cps = [None] * _NS2
        for s in range(_LOOKAHEAD):
            cps[s] = gather(s)
        sc_cps = [None] * _NS2
        cp_n = None
        for s in range(_NS2):
            if s > 0:
                sc_cps[s - 1].wait()
            if s + _LOOKAHEAD < _NS2:
                cps[s + _LOOKAHEAD] = gather(s + _LOOKAHEAD)
            if prefetch and s == 0:
                cp_idx.wait()
                compute_idx8(nidxv, nidx8)
                cp_n = pltpu.async_copy(n0_r.at[nidx8], nbbuf, sem_n0)
            if prefetch and s == 6:
                cp_n.wait()
                transpose_nb(nidxv, nnbt0)
                cp_n = pltpu.async_copy(n1_r.at[nidx8], nbbuf, sem_n0)
            if prefetch and s == 12:
                cp_n.wait()
                transpose_nb(nidxv, nnbt1)
            cps[s].wait()
            r = 0 if s < _DEG else 1
            sc_cps[s] = pltpu.async_copy(
                rbufs[s % _NB], accs.at[sid, r].at[perm], ssems[s % _NB],
                add=(s % _DEG != 0))
        sc_cps[_NS2 - 1].wait()
        cp_o0 = pltpu.async_copy(accs.at[sid, 0], sum0_r.at[pl.ds(rowbase, _C)],
                                 sem_o)
        cp_o1 = pltpu.async_copy(accs.at[sid, 1], sum1_r.at[pl.ds(rowbase, _C)],
                                 sem_o)
        cp_self.wait()
        cp_os = pltpu.async_copy(selfbuf, selfs_r.at[pl.ds(rowbase, _C)], sem_o)
        cp_o0.wait()
        cp_o1.wait()
        cp_os.wait()

    bufa = (idxa, nbta0, nbta1)
    bufb = (idxb, nbtb0, nbtb1)
    head(0, idxa, idx8a, nbta0, nbta1)

    def pair_body(m, carry):
        k = 2 * m
        steps(k, *bufa, idxb, idx8b, nbtb0, nbtb1, prefetch=True)
        steps(k + 1, *bufb, idxa, idx8a, nbta0, nbta1, prefetch=True)
        return carry

    lax.fori_loop(0, (_NCH - 1) // 2, pair_body, 0)
    steps(_NCH - 1, *bufa, idxb, idx8b, nbtb0, nbtb1, prefetch=False)


_sc_gather = pl.kernel(
    _sc_body,
    out_type=(
        jax.ShapeDtypeStruct((_B, _D), jnp.float32),
        jax.ShapeDtypeStruct((_B, _D), jnp.float32),
        jax.ShapeDtypeStruct((_B, _D), jnp.float32),
    ),
    mesh=plsc.VectorSubcoreMesh(
        core_axis_name="c", subcore_axis_name="s",
        num_cores=_NC, num_subcores=_NS),
    compiler_params=pltpu.CompilerParams(needs_layout_passes=False),
    scratch_types=[
        pltpu.VMEM((_C,), jnp.int32),
        pltpu.VMEM((_C,), jnp.int32),
        pltpu.VMEM((_C,), jnp.int32),
        pltpu.VMEM((_C,), jnp.int32),
        pltpu.VMEM((_C,), jnp.int32),
        pltpu.VMEM((_C, _D), jnp.int32),
        pltpu.VMEM((_DEG, _C), jnp.int32),
        pltpu.VMEM((_DEG, _C), jnp.int32),
        pltpu.VMEM((_DEG, _C), jnp.int32),
        pltpu.VMEM((_DEG, _C), jnp.int32),
        pltpu.VMEM((_C, _D), jnp.float32),
        tuple(pltpu.VMEM((_C, _D), jnp.float32) for _ in range(4)),
        pltpu.VMEM_SHARED((_NS, 2, _C, _D), jnp.float32),
        tuple(pltpu.SemaphoreType.DMA for _ in range(4)),
        tuple(pltpu.SemaphoreType.DMA for _ in range(4)),
        pltpu.SemaphoreType.DMA,
        pltpu.SemaphoreType.DMA,
        pltpu.SemaphoreType.DMA,
        pltpu.SemaphoreType.DMA,
    ],
)


def _mlp_body(xs_ref, x0_ref, x1_ref, w1_ref, b1_ref, w2_ref, b2_ref, o_ref):
    dot = functools.partial(jnp.dot, preferred_element_type=jnp.float32,
                            precision=lax.Precision.DEFAULT)
    xs = xs_ref[...]
    x0 = x0_ref[...] * (1.0 / _DEG)
    x1 = x1_ref[...] * (1.0 / _DEG)
    w1 = w1_ref[...]
    acc = dot(xs, w1[0:_D])
    acc = acc + dot(x0, w1[_D:2 * _D])
    acc = acc + dot(x1, w1[2 * _D:3 * _D])
    h = jnp.tanh(acc + b1_ref[...])
    o_ref[...] = dot(h, w2_ref[...]) + b2_ref[...]


_RB = 256


def _mlp(xs, x0, x1, W1, b1, W2, b2):
    grid = ((_B + _RB - 1) // _RB,)
    blk = lambda i: (i, 0)
    full = lambda i: (0, 0)
    return pl.pallas_call(
        _mlp_body,
        grid=grid,
        in_specs=[
            pl.BlockSpec((_RB, _D), blk),
            pl.BlockSpec((_RB, _D), blk),
            pl.BlockSpec((_RB, _D), blk),
            pl.BlockSpec((3 * _D, _D), full),
            pl.BlockSpec((1, _D), full),
            pl.BlockSpec((_D, _E), full),
            pl.BlockSpec((1, _E), full),
        ],
        out_specs=pl.BlockSpec((_RB, _E), blk),
        out_shape=jax.ShapeDtypeStruct((_B, _E), jnp.float32),
    )(xs, x0, x1, W1, b1, W2, b2)


def kernel(nodes, features, neigh_idx_0, neigh_idx_1, W1, b1, W2, b2):
    n0r = neigh_idx_0.reshape(_N * _DEG // _D, _D)
    n1r = neigh_idx_1.reshape(_N * _DEG // _D, _D)
    selfs, sum0, sum1 = _sc_gather(nodes, features, n0r, n1r)
    return _mlp(selfs, sum0, sum1, W1, b1.reshape(1, _D), W2, b2.reshape(1, _E))
